# baseline (speedup 1.0000x reference)
import functools

import jax
import jax.numpy as jnp
from jax import lax
from jax.experimental import pallas as pl
from jax.experimental.pallas import tpu as pltpu
from jax.experimental.pallas import tpu_sc as plsc

N = 10000
NP = 10240
E = 320000
D = 128

NC = 2
NS = 16
K = 40
EPC = E // NC
EPT = EPC // NS
NCHUNK = EPT // K
RPT = NP // NS

_MESH = plsc.VectorSubcoreMesh(core_axis_name="c", subcore_axis_name="s")


KD = 80
DNB = 5
DGROUP = (EPT // KD) // DNB


@functools.partial(
    pl.kernel,
    out_type=jax.ShapeDtypeStruct((NC, 2, NP), jnp.float32),
    mesh=_MESH,
    scratch_types=[
        pltpu.VMEM_SHARED((NP,), jnp.float32),
        pltpu.VMEM_SHARED((NP,), jnp.float32),
        [pltpu.VMEM((KD,), jnp.int32)] * DNB,
        [pltpu.VMEM((KD,), jnp.int32)] * DNB,
        [pltpu.SemaphoreType.DMA] * DNB,
        pltpu.VMEM((RPT,), jnp.float32),
        pltpu.VMEM((KD,), jnp.float32),
    ],
)
def _deg_kernel(src_hbm, dst_hbm, out_hbm, acc_o, acc_i, sidx, didx, isem, zbuf, ones):
    c = lax.axis_index("c")
    s = lax.axis_index("s")

    def fill_z(j, carry):
        zbuf[pl.ds(j * 16, 16)] = jnp.zeros((16,), jnp.float32)
        return carry

    lax.fori_loop(0, RPT // 16, fill_z, 0)

    def fill_o(j, carry):
        ones[pl.ds(j * 16, 16)] = jnp.full((16,), 1.0, jnp.float32)
        return carry

    lax.fori_loop(0, KD // 16, fill_o, 0)

    pltpu.sync_copy(zbuf, acc_o.at[pl.ds(s * RPT, RPT)])
    pltpu.sync_copy(zbuf, acc_i.at[pl.ds(s * RPT, RPT)])
    plsc.subcore_barrier()

    tile_base = c * EPC + s * EPT

    def load_idx(i, b):
        base = pl.multiple_of(tile_base + i * KD, 8)
        pltpu.async_copy(src_hbm.at[pl.ds(base, KD)], sidx[b], isem[b])
        pltpu.async_copy(dst_hbm.at[pl.ds(base, KD)], didx[b], isem[b])

    def wait_idx(b):
        pltpu.make_async_copy(src_hbm.at[pl.ds(0, KD)], sidx[b], isem[b]).wait()
        pltpu.make_async_copy(dst_hbm.at[pl.ds(0, KD)], didx[b], isem[b]).wait()

    for b in range(DNB):
        load_idx(b, b)

    def group(g, carry):
        for b in range(DNB):
            wait_idx(b)
            pltpu.sync_copy(ones, acc_o.at[sidx[b]], add=True)
            pltpu.sync_copy(ones, acc_i.at[didx[b]], add=True)

            @pl.when(g < DGROUP - 1)
            def _():
                load_idx((g + 1) * DNB + b, b)

        return carry

    lax.fori_loop(0, DGROUP, group, 0)
    plsc.subcore_barrier()

    off = pl.multiple_of(s * RPT, 8)
    pltpu.sync_copy(acc_o.at[pl.ds(off, RPT)], out_hbm.at[c, 0, pl.ds(off, RPT)])
    pltpu.sync_copy(acc_i.at[pl.ds(off, RPT)], out_hbm.at[c, 1, pl.ds(off, RPT)])


NBUF = 5
NGROUP = NCHUNK // NBUF


@functools.partial(
    pl.kernel,
    out_type=jax.ShapeDtypeStruct((NC, NP, D), jnp.float32),
    mesh=_MESH,
    scratch_types=[
        pltpu.VMEM_SHARED((NP, D), jnp.float32),
        [[pltpu.VMEM((K,), jnp.int32)] * NBUF] * 2,
        [[pltpu.VMEM((K,), jnp.int32)] * NBUF] * 2,
        [pltpu.VMEM((K, D), jnp.float32)] * NBUF,
        [pltpu.SemaphoreType.DMA] * NBUF,
        [pltpu.SemaphoreType.DMA] * NBUF,
        [[pltpu.SemaphoreType.DMA] * NBUF] * 2,
    ],
)
def _agg_kernel(y_hbm, src_hbm, dst_hbm, out_hbm, acc, sidx, didx, rowss,
                gsem, ssem, isem):
    c = lax.axis_index("c")
    s = lax.axis_index("s")

    def fill_z(r, carry):
        for cb in range(D // 16):
            rowss[0][r, pl.ds(cb * 16, 16)] = jnp.zeros((16,), jnp.float32)
        return carry

    lax.fori_loop(0, K, fill_z, 0)

    def zero_acc(j, carry):
        pltpu.sync_copy(rowss[0], acc.at[pl.ds(s * RPT + j * K, K)])
        return carry

    lax.fori_loop(0, RPT // K, zero_acc, 0)
    plsc.subcore_barrier()

    tile_base = c * EPC + s * EPT

    def load_idx(i, p, b):
        base = pl.multiple_of(tile_base + i * K, 8)
        pltpu.async_copy(src_hbm.at[pl.ds(base, K)], sidx[p][b], isem[p][b])
        pltpu.async_copy(dst_hbm.at[pl.ds(base, K)], didx[p][b], isem[p][b])

    def wait_idx(p, b):
        pltpu.make_async_copy(src_hbm.at[pl.ds(0, K)], sidx[p][b], isem[p][b]).wait()
        pltpu.make_async_copy(dst_hbm.at[pl.ds(0, K)], didx[p][b], isem[p][b]).wait()

    for b in range(NBUF):
        load_idx(b, 0, b)
    for b in range(NBUF):
        wait_idx(0, b)
        pltpu.async_copy(y_hbm.at[sidx[0][b]], rowss[b], gsem[b])
        load_idx(NBUF + b, 1, b)

    def dgroup(gg, carry):
        for p in range(2):
            g = gg * 2 + p
            p1 = 1 - p
            for b in range(NBUF):
                pltpu.make_async_copy(y_hbm.at[sidx[p][b]], rowss[b], gsem[b]).wait()
                pltpu.async_copy(rowss[b], acc.at[didx[p][b]], ssem[b], add=True)
            for b in range(NBUF):
                pltpu.make_async_copy(
                    rowss[b], acc.at[didx[p][b]], ssem[b]
                ).wait()

                @pl.when(g < NGROUP - 1)
                def _():
                    wait_idx(p1, b)
                    pltpu.async_copy(y_hbm.at[sidx[p1][b]], rowss[b], gsem[b])

                @pl.when(g < NGROUP - 2)
                def _():
                    load_idx((g + 2) * NBUF + b, p, b)

        return carry

    lax.fori_loop(0, NGROUP // 2, dgroup, 0)
    plsc.subcore_barrier()

    off = pl.multiple_of(s * RPT, 8)
    pltpu.sync_copy(acc.at[pl.ds(off, RPT)], out_hbm.at[c, pl.ds(off, RPT)])


BN = 2048


def _norms(deg):
    ns = lax.rsqrt(jnp.maximum(deg[0] + deg[2], 1.0))
    nd = lax.rsqrt(jnp.maximum(deg[1] + deg[3], 1.0))
    return ns, nd


def _mm1_body(deg_ref, x_ref, w_ref, o_ref):
    ns, _ = _norms(deg_ref[...])
    o_ref[...] = jnp.dot(
        x_ref[...] * ns[:, None], w_ref[...], preferred_element_type=jnp.float32
    )


def _mid_body(deg_ref, p_ref, b_ref, w_ref, o_ref):
    ns, nd = _norms(deg_ref[...])
    agg = p_ref[0] + p_ref[1]
    h = jnp.maximum(agg * nd[:, None] + b_ref[...], 0.0)
    o_ref[...] = jnp.dot(
        h * ns[:, None], w_ref[...], preferred_element_type=jnp.float32
    )


def _fin_body(deg_ref, q_ref, b_ref, o_ref):
    _, nd = _norms(deg_ref[...])
    o_ref[...] = (q_ref[0] + q_ref[1]) * nd[:, None] + b_ref[...]


_DEG_SPEC = pl.BlockSpec((4, BN), lambda i: (0, i))
_ROW_SPEC = pl.BlockSpec((BN, D), lambda i: (i, 0))
_P_SPEC = pl.BlockSpec((2, BN, D), lambda i: (0, i, 0))
_W_SPEC = pl.BlockSpec((D, D), lambda i: (0, 0))
_B_SPEC = pl.BlockSpec((1, D), lambda i: (0, 0))
_OSHAPE = jax.ShapeDtypeStruct((NP, D), jnp.float32)


def _tc_mm1(deg, x, w):
    return pl.pallas_call(
        _mm1_body,
        grid=(NP // BN,),
        in_specs=[_DEG_SPEC, _ROW_SPEC, _W_SPEC],
        out_specs=_ROW_SPEC,
        out_shape=_OSHAPE,
    )(deg, x, w)


def _tc_mid(deg, p, b, w):
    return pl.pallas_call(
        _mid_body,
        grid=(NP // BN,),
        in_specs=[_DEG_SPEC, _P_SPEC, _B_SPEC, _W_SPEC],
        out_specs=_ROW_SPEC,
        out_shape=_OSHAPE,
    )(deg, p, b, w)


def _tc_fin(deg, q, b):
    return pl.pallas_call(
        _fin_body,
        grid=(NP // BN,),
        in_specs=[_DEG_SPEC, _P_SPEC, _B_SPEC],
        out_specs=_ROW_SPEC,
        out_shape=_OSHAPE,
    )(deg, q, b)


def kernel(x, edge_index, W1, b1, W2, b2):
    src = edge_index[0]
    dst = edge_index[1]

    deg = _deg_kernel(src, dst)
    deg = deg.reshape(2 * 2, NP)

    xp = jnp.zeros((NP, D), jnp.float32).at[:N].set(x)
    b1r = b1.reshape(1, D)
    b2r = b2.reshape(1, D)

    y1 = _tc_mm1(deg, xp, W1)
    p1 = _agg_kernel(y1, src, dst)
    y2 = _tc_mid(deg, p1, b1r, W2)
    p2 = _agg_kernel(y2, src, dst)
    out = _tc_fin(deg, p2, b2r)
    return out[:N]


# speedup vs baseline: 1.1510x; 1.1510x over previous
import functools

import jax
import jax.numpy as jnp
from jax import lax
from jax.experimental import pallas as pl
from jax.experimental.pallas import tpu as pltpu
from jax.experimental.pallas import tpu_sc as plsc

N = 10000
NP = 10240
E = 320000
D = 128

NC = 2
NS = 16
K = 40
EPC = E // NC
EPT = EPC // NS
NCHUNK = EPT // K
RPT = NP // NS

_MESH = plsc.VectorSubcoreMesh(core_axis_name="c", subcore_axis_name="s")


KD = 80
DNB = 5
DGROUP = (EPT // KD) // DNB


@functools.partial(
    pl.kernel,
    out_type=jax.ShapeDtypeStruct((NC, 2, NP), jnp.float32),
    mesh=_MESH,
    scratch_types=[
        pltpu.VMEM_SHARED((NP,), jnp.float32),
        pltpu.VMEM_SHARED((NP,), jnp.float32),
        [pltpu.VMEM((KD,), jnp.int32)] * DNB,
        [pltpu.VMEM((KD,), jnp.int32)] * DNB,
        [pltpu.SemaphoreType.DMA] * DNB,
        pltpu.VMEM((RPT,), jnp.float32),
        pltpu.VMEM((KD,), jnp.float32),
    ],
)
def _deg_kernel(src_hbm, dst_hbm, out_hbm, acc_o, acc_i, sidx, didx, isem, zbuf, ones):
    c = lax.axis_index("c")
    s = lax.axis_index("s")

    def fill_z(j, carry):
        zbuf[pl.ds(j * 16, 16)] = jnp.zeros((16,), jnp.float32)
        return carry

    lax.fori_loop(0, RPT // 16, fill_z, 0)

    def fill_o(j, carry):
        ones[pl.ds(j * 16, 16)] = jnp.full((16,), 1.0, jnp.float32)
        return carry

    lax.fori_loop(0, KD // 16, fill_o, 0)

    pltpu.sync_copy(zbuf, acc_o.at[pl.ds(s * RPT, RPT)])
    pltpu.sync_copy(zbuf, acc_i.at[pl.ds(s * RPT, RPT)])
    plsc.subcore_barrier()

    tile_base = c * EPC + s * EPT

    def load_idx(i, b):
        base = pl.multiple_of(tile_base + i * KD, 8)
        pltpu.async_copy(src_hbm.at[pl.ds(base, KD)], sidx[b], isem[b])
        pltpu.async_copy(dst_hbm.at[pl.ds(base, KD)], didx[b], isem[b])

    def wait_idx(b):
        pltpu.make_async_copy(src_hbm.at[pl.ds(0, KD)], sidx[b], isem[b]).wait()
        pltpu.make_async_copy(dst_hbm.at[pl.ds(0, KD)], didx[b], isem[b]).wait()

    for b in range(DNB):
        load_idx(b, b)

    def group(g, carry):
        for b in range(DNB):
            wait_idx(b)
            pltpu.sync_copy(ones, acc_o.at[sidx[b]], add=True)
            pltpu.sync_copy(ones, acc_i.at[didx[b]], add=True)

            @pl.when(g < DGROUP - 1)
            def _():
                load_idx((g + 1) * DNB + b, b)

        return carry

    lax.fori_loop(0, DGROUP, group, 0)
    plsc.subcore_barrier()

    off = pl.multiple_of(s * RPT, 8)
    pltpu.sync_copy(acc_o.at[pl.ds(off, RPT)], out_hbm.at[c, 0, pl.ds(off, RPT)])
    pltpu.sync_copy(acc_i.at[pl.ds(off, RPT)], out_hbm.at[c, 1, pl.ds(off, RPT)])


NBUF = 5
NGROUP = NCHUNK // NBUF


@functools.partial(
    pl.kernel,
    out_type=jax.ShapeDtypeStruct((NC, NP, D), jnp.float32),
    mesh=_MESH,
    scratch_types=[
        pltpu.VMEM_SHARED((NP, D), jnp.float32),
        [[pltpu.VMEM((K,), jnp.int32)] * NBUF] * 2,
        [[pltpu.VMEM((K,), jnp.int32)] * NBUF] * 2,
        [pltpu.VMEM((K, D), jnp.float32)] * NBUF,
        [pltpu.SemaphoreType.DMA] * NBUF,
        [[pltpu.SemaphoreType.DMA] * NBUF] * 2,
    ],
)
def _agg_kernel(y_hbm, src_hbm, dst_hbm, out_hbm, acc, sidx, didx, rowss,
                gsem, isem):
    c = lax.axis_index("c")
    s = lax.axis_index("s")

    def fill_z(r, carry):
        for cb in range(D // 16):
            rowss[0][r, pl.ds(cb * 16, 16)] = jnp.zeros((16,), jnp.float32)
        return carry

    lax.fori_loop(0, K, fill_z, 0)

    def zero_acc(j, carry):
        pltpu.sync_copy(rowss[0], acc.at[pl.ds(s * RPT + j * K, K)])
        return carry

    lax.fori_loop(0, RPT // K, zero_acc, 0)
    plsc.subcore_barrier()

    tile_base = c * EPC + s * EPT

    def load_idx(i, p, b):
        base = pl.multiple_of(tile_base + i * K, 8)
        pltpu.async_copy(src_hbm.at[pl.ds(base, K)], sidx[p][b], isem[p][b])
        pltpu.async_copy(dst_hbm.at[pl.ds(base, K)], didx[p][b], isem[p][b])

    def wait_idx(p, b):
        pltpu.make_async_copy(src_hbm.at[pl.ds(0, K)], sidx[p][b], isem[p][b]).wait()
        pltpu.make_async_copy(dst_hbm.at[pl.ds(0, K)], didx[p][b], isem[p][b]).wait()

    for b in range(NBUF):
        load_idx(b, 0, b)
    for b in range(NBUF):
        wait_idx(0, b)
        pltpu.async_copy(y_hbm.at[sidx[0][b]], rowss[b], gsem[b])
        load_idx(NBUF + b, 1, b)

    def dgroup(gg, carry):
        for p in range(2):
            g = gg * 2 + p
            p1 = 1 - p
            for b in range(NBUF):
                pltpu.make_async_copy(y_hbm.at[sidx[p][b]], rowss[b], gsem[b]).wait()
                pltpu.sync_copy(rowss[b], acc.at[didx[p][b]], add=True)

                @pl.when(g < NGROUP - 1)
                def _():
                    wait_idx(p1, b)
                    pltpu.async_copy(y_hbm.at[sidx[p1][b]], rowss[b], gsem[b])

                @pl.when(g < NGROUP - 2)
                def _():
                    load_idx((g + 2) * NBUF + b, p, b)

        return carry

    lax.fori_loop(0, NGROUP // 2, dgroup, 0)
    plsc.subcore_barrier()

    off = pl.multiple_of(s * RPT, 8)
    pltpu.sync_copy(acc.at[pl.ds(off, RPT)], out_hbm.at[c, pl.ds(off, RPT)])


BN = 2000


def _norms(degt):
    ns = lax.rsqrt(jnp.maximum(degt[:, 0] + degt[:, 2], 1.0))
    nd = lax.rsqrt(jnp.maximum(degt[:, 1] + degt[:, 3], 1.0))
    return ns, nd


def _mm1_body(deg_ref, x_ref, w_ref, o_ref):
    ns, _ = _norms(deg_ref[...])
    o_ref[...] = jnp.dot(
        x_ref[...] * ns[:, None], w_ref[...], preferred_element_type=jnp.float32
    )


def _mid_body(deg_ref, p_ref, b_ref, w_ref, o_ref):
    ns, nd = _norms(deg_ref[...])
    agg = p_ref[0] + p_ref[1]
    h = jnp.maximum(agg * nd[:, None] + b_ref[...], 0.0)
    o_ref[...] = jnp.dot(
        h * ns[:, None], w_ref[...], preferred_element_type=jnp.float32
    )


def _fin_body(deg_ref, q_ref, b_ref, o_ref):
    _, nd = _norms(deg_ref[...])
    o_ref[...] = (q_ref[0] + q_ref[1]) * nd[:, None] + b_ref[...]


_DEG_SPEC = pl.BlockSpec((BN, 4), lambda i: (i, 0))
_ROW_SPEC = pl.BlockSpec((BN, D), lambda i: (i, 0))
_P_SPEC = pl.BlockSpec((2, BN, D), lambda i: (0, i, 0))
_W_SPEC = pl.BlockSpec((D, D), lambda i: (0, 0))
_B_SPEC = pl.BlockSpec((1, D), lambda i: (0, 0))
_OSHAPE = jax.ShapeDtypeStruct((N, D), jnp.float32)


def _tc_mm1(degt, x, w):
    return pl.pallas_call(
        _mm1_body,
        grid=(N // BN,),
        in_specs=[_DEG_SPEC, _ROW_SPEC, _W_SPEC],
        out_specs=_ROW_SPEC,
        out_shape=_OSHAPE,
    )(degt, x, w)


def _tc_mid(degt, p, b, w):
    return pl.pallas_call(
        _mid_body,
        grid=(N // BN,),
        in_specs=[_DEG_SPEC, _P_SPEC, _B_SPEC, _W_SPEC],
        out_specs=_ROW_SPEC,
        out_shape=_OSHAPE,
    )(degt, p, b, w)


def _tc_fin(degt, q, b):
    return pl.pallas_call(
        _fin_body,
        grid=(N // BN,),
        in_specs=[_DEG_SPEC, _P_SPEC, _B_SPEC],
        out_specs=_ROW_SPEC,
        out_shape=_OSHAPE,
    )(degt, q, b)


def kernel(x, edge_index, W1, b1, W2, b2):
    src = edge_index[0]
    dst = edge_index[1]

    deg = _deg_kernel(src, dst)
    degt = deg.reshape(2 * 2, NP)[:, :N].T

    b1r = b1.reshape(1, D)
    b2r = b2.reshape(1, D)

    y1 = _tc_mm1(degt, x, W1)
    p1 = _agg_kernel(y1, src, dst)
    y2 = _tc_mid(degt, p1, b1r, W2)
    p2 = _agg_kernel(y2, src, dst)
    return _tc_fin(degt, p2, b2r)


# speedup vs baseline: 1.1874x; 1.0316x over previous
import functools

import jax
import jax.numpy as jnp
from jax import lax
from jax.experimental import pallas as pl
from jax.experimental.pallas import tpu as pltpu
from jax.experimental.pallas import tpu_sc as plsc

N = 10000
NP = 10240
E = 320000
D = 128

NC = 2
NS = 16
K = 40
EPC = E // NC
EPT = EPC // NS
NCHUNK = EPT // K
RPT = NP // NS

_MESH = plsc.VectorSubcoreMesh(core_axis_name="c", subcore_axis_name="s")


KD = 128
DNB = 6
DNC = EPT // KD
DTAIL = EPT - DNC * KD
DGROUP = DNC // DNB


@functools.partial(
    pl.kernel,
    out_type=jax.ShapeDtypeStruct((NC, 2, NP), jnp.float32),
    mesh=_MESH,
    scratch_types=[
        pltpu.VMEM_SHARED((NP,), jnp.float32),
        pltpu.VMEM_SHARED((NP,), jnp.float32),
        [pltpu.VMEM((KD,), jnp.int32)] * DNB,
        [pltpu.VMEM((KD,), jnp.int32)] * DNB,
        pltpu.VMEM((DTAIL,), jnp.int32),
        pltpu.VMEM((DTAIL,), jnp.int32),
        [pltpu.SemaphoreType.DMA] * DNB,
        pltpu.VMEM((RPT,), jnp.float32),
        pltpu.VMEM((KD,), jnp.float32),
    ],
)
def _deg_kernel(src_hbm, dst_hbm, out_hbm, acc_o, acc_i, sidx, didx,
                tsidx, tdidx, isem, zbuf, ones):
    c = lax.axis_index("c")
    s = lax.axis_index("s")

    def fill_z(j, carry):
        zbuf[pl.ds(j * 16, 16)] = jnp.zeros((16,), jnp.float32)
        return carry

    lax.fori_loop(0, RPT // 16, fill_z, 0)

    def fill_o(j, carry):
        ones[pl.ds(j * 16, 16)] = jnp.full((16,), 1.0, jnp.float32)
        return carry

    lax.fori_loop(0, KD // 16, fill_o, 0)

    pltpu.sync_copy(zbuf, acc_o.at[pl.ds(s * RPT, RPT)])
    pltpu.sync_copy(zbuf, acc_i.at[pl.ds(s * RPT, RPT)])
    plsc.subcore_barrier()

    tile_base = c * EPC + s * EPT

    def load_idx(i, b):
        base = pl.multiple_of(tile_base + i * KD, 8)
        pltpu.async_copy(src_hbm.at[pl.ds(base, KD)], sidx[b], isem[b])
        pltpu.async_copy(dst_hbm.at[pl.ds(base, KD)], didx[b], isem[b])

    def wait_idx(b):
        pltpu.make_async_copy(src_hbm.at[pl.ds(0, KD)], sidx[b], isem[b]).wait()
        pltpu.make_async_copy(dst_hbm.at[pl.ds(0, KD)], didx[b], isem[b]).wait()

    for b in range(DNB):
        load_idx(b, b)

    def group(g, carry):
        for b in range(DNB):
            wait_idx(b)
            pltpu.sync_copy(ones, acc_o.at[sidx[b]], add=True)
            pltpu.sync_copy(ones, acc_i.at[didx[b]], add=True)

            @pl.when(g < DGROUP - 1)
            def _():
                load_idx((g + 1) * DNB + b, b)

        return carry

    lax.fori_loop(0, DGROUP, group, 0)

    tbase = pl.multiple_of(tile_base + DNC * KD, 8)
    pltpu.sync_copy(src_hbm.at[pl.ds(tbase, DTAIL)], tsidx)
    pltpu.sync_copy(dst_hbm.at[pl.ds(tbase, DTAIL)], tdidx)
    pltpu.sync_copy(ones.at[pl.ds(0, DTAIL)], acc_o.at[tsidx], add=True)
    pltpu.sync_copy(ones.at[pl.ds(0, DTAIL)], acc_i.at[tdidx], add=True)

    plsc.subcore_barrier()

    off = pl.multiple_of(s * RPT, 8)
    pltpu.sync_copy(acc_o.at[pl.ds(off, RPT)], out_hbm.at[c, 0, pl.ds(off, RPT)])
    pltpu.sync_copy(acc_i.at[pl.ds(off, RPT)], out_hbm.at[c, 1, pl.ds(off, RPT)])


NBUF = 5
NGROUP = NCHUNK // NBUF


@functools.partial(
    pl.kernel,
    out_type=jax.ShapeDtypeStruct((NC, NP, D), jnp.float32),
    mesh=_MESH,
    scratch_types=[
        pltpu.VMEM_SHARED((NP, D), jnp.float32),
        [[pltpu.VMEM((K,), jnp.int32)] * NBUF] * 2,
        [[pltpu.VMEM((K,), jnp.int32)] * NBUF] * 2,
        [pltpu.VMEM((K, D), jnp.float32)] * NBUF,
        [pltpu.SemaphoreType.DMA] * NBUF,
        [[pltpu.SemaphoreType.DMA] * NBUF] * 2,
    ],
)
def _agg_kernel(y_hbm, src_hbm, dst_hbm, out_hbm, acc, sidx, didx, rowss,
                gsem, isem):
    c = lax.axis_index("c")
    s = lax.axis_index("s")

    def fill_z(r, carry):
        for cb in range(D // 16):
            rowss[0][r, pl.ds(cb * 16, 16)] = jnp.zeros((16,), jnp.float32)
        return carry

    lax.fori_loop(0, K, fill_z, 0)

    def zero_acc(j, carry):
        pltpu.sync_copy(rowss[0], acc.at[pl.ds(s * RPT + j * K, K)])
        return carry

    lax.fori_loop(0, RPT // K, zero_acc, 0)
    plsc.subcore_barrier()

    tile_base = c * EPC + s * EPT

    def load_idx(i, p, b):
        base = pl.multiple_of(tile_base + i * K, 8)
        pltpu.async_copy(src_hbm.at[pl.ds(base, K)], sidx[p][b], isem[p][b])
        pltpu.async_copy(dst_hbm.at[pl.ds(base, K)], didx[p][b], isem[p][b])

    def wait_idx(p, b):
        pltpu.make_async_copy(src_hbm.at[pl.ds(0, K)], sidx[p][b], isem[p][b]).wait()
        pltpu.make_async_copy(dst_hbm.at[pl.ds(0, K)], didx[p][b], isem[p][b]).wait()

    for b in range(NBUF):
        load_idx(b, 0, b)
    for b in range(NBUF):
        wait_idx(0, b)
        pltpu.async_copy(y_hbm.at[sidx[0][b]], rowss[b], gsem[b])
        load_idx(NBUF + b, 1, b)

    def dgroup(gg, carry):
        for p in range(2):
            g = gg * 2 + p
            p1 = 1 - p
            for b in range(NBUF):
                pltpu.make_async_copy(y_hbm.at[sidx[p][b]], rowss[b], gsem[b]).wait()
                pltpu.sync_copy(rowss[b], acc.at[didx[p][b]], add=True)

                @pl.when(g < NGROUP - 1)
                def _():
                    wait_idx(p1, b)
                    pltpu.async_copy(y_hbm.at[sidx[p1][b]], rowss[b], gsem[b])

                @pl.when(g < NGROUP - 2)
                def _():
                    load_idx((g + 2) * NBUF + b, p, b)

        return carry

    lax.fori_loop(0, NGROUP // 2, dgroup, 0)
    plsc.subcore_barrier()

    off = pl.multiple_of(s * RPT, 8)
    pltpu.sync_copy(acc.at[pl.ds(off, RPT)], out_hbm.at[c, pl.ds(off, RPT)])


BN = 2048


def _norms(deg):
    ns = lax.rsqrt(jnp.maximum(deg[0] + deg[2], 1.0))
    nd = lax.rsqrt(jnp.maximum(deg[1] + deg[3], 1.0))
    return ns, nd


def _mm1_body(deg_ref, x_ref, w_ref, o_ref):
    ns, _ = _norms(deg_ref[...])
    o_ref[...] = jnp.dot(
        x_ref[...] * ns[:, None], w_ref[...], preferred_element_type=jnp.float32
    )


def _mid_body(deg_ref, p_ref, b_ref, w_ref, o_ref):
    ns, nd = _norms(deg_ref[...])
    agg = p_ref[0] + p_ref[1]
    h = jnp.maximum(agg * nd[:, None] + b_ref[...], 0.0)
    o_ref[...] = jnp.dot(
        h * ns[:, None], w_ref[...], preferred_element_type=jnp.float32
    )


def _fin_body(deg_ref, q_ref, b_ref, o_ref):
    _, nd = _norms(deg_ref[...])
    o_ref[...] = (q_ref[0] + q_ref[1]) * nd[:, None] + b_ref[...]


_DEG_SPEC = pl.BlockSpec((4, BN), lambda i: (0, i))
_ROW_SPEC = pl.BlockSpec((BN, D), lambda i: (i, 0))
_P_SPEC = pl.BlockSpec((2, BN, D), lambda i: (0, i, 0))
_W_SPEC = pl.BlockSpec((D, D), lambda i: (0, 0))
_B_SPEC = pl.BlockSpec((1, D), lambda i: (0, 0))
_OSHAPE = jax.ShapeDtypeStruct((NP, D), jnp.float32)


def _tc_mm1(deg, x, w):
    return pl.pallas_call(
        _mm1_body,
        grid=(NP // BN,),
        in_specs=[_DEG_SPEC, _ROW_SPEC, _W_SPEC],
        out_specs=_ROW_SPEC,
        out_shape=_OSHAPE,
    )(deg, x, w)


def _tc_mid(deg, p, b, w):
    return pl.pallas_call(
        _mid_body,
        grid=(NP // BN,),
        in_specs=[_DEG_SPEC, _P_SPEC, _B_SPEC, _W_SPEC],
        out_specs=_ROW_SPEC,
        out_shape=_OSHAPE,
    )(deg, p, b, w)


def _tc_fin(deg, q, b):
    return pl.pallas_call(
        _fin_body,
        grid=(NP // BN,),
        in_specs=[_DEG_SPEC, _P_SPEC, _B_SPEC],
        out_specs=_ROW_SPEC,
        out_shape=_OSHAPE,
    )(deg, q, b)


def kernel(x, edge_index, W1, b1, W2, b2):
    src = edge_index[0]
    dst = edge_index[1]

    deg = _deg_kernel(src, dst)
    deg = deg.reshape(2 * 2, NP)

    xp = jnp.zeros((NP, D), jnp.float32).at[:N].set(x)
    b1r = b1.reshape(1, D)
    b2r = b2.reshape(1, D)

    y1 = _tc_mm1(deg, xp, W1)
    p1 = _agg_kernel(y1, src, dst)
    y2 = _tc_mid(deg, p1, b1r, W2)
    p2 = _agg_kernel(y2, src, dst)
    out = _tc_fin(deg, p2, b2r)
    return out[:N]
